# baseline (speedup 1.0000x reference)
import jax
import jax.numpy as jnp
from jax import lax
from jax.experimental import pallas as pl
from jax.experimental.pallas import tpu as pltpu

B, C, H, W, N = 16, 80, 100, 100, 32
LAMBDA_L1, LAMBDA_GIOU, POS_IOU_THR = 1.0, 2.0, 0.3
R_LO, R_HI = 2, 1
G = 8


def _reduce2(x, op):
    r = op(x, axis=0, keepdims=True)
    return op(r, axis=1, keepdims=True)


def _giou(x0, y0, x1, y1, gx0, gy0, gx1, gy1):
    area_a = (x1 - x0) * (y1 - y0)
    area_b = (gx1 - gx0) * (gy1 - gy0)
    ltx = jnp.maximum(x0, gx0)
    lty = jnp.maximum(y0, gy0)
    rbx = jnp.minimum(x1, gx1)
    rby = jnp.minimum(y1, gy1)
    iw = jnp.maximum(rbx - ltx, 0.0)
    ih = jnp.maximum(rby - lty, 0.0)
    inter = iw * ih
    union = area_a + area_b - inter
    iou = inter / union
    cx = jnp.minimum(x0, gx0)
    cy = jnp.minimum(y0, gy0)
    dx = jnp.maximum(x1, gx1)
    dy = jnp.maximum(y1, gy1)
    area_c = (dx - cx) * (dy - cy)
    return iou - (area_c - union) / area_c


def _one_gt(pred_ref, gx0, gy0, gx1, gy1):
    x0 = pred_ref[0, 0, 0::4, :]
    y0 = pred_ref[0, 0, 1::4, :]
    x1 = pred_ref[0, 0, 2::4, :]
    y1 = pred_ref[0, 0, 3::4, :]

    g = _giou(x0, y0, x1, y1, gx0, gy0, gx1, gy1)

    m = _reduce2(g, jnp.max)
    rows_f = lax.broadcasted_iota(jnp.int32, (H, W), 0).astype(jnp.float32)
    cols_f = lax.broadcasted_iota(jnp.int32, (H, W), 1).astype(jnp.float32)
    key = rows_f * 128.0 + cols_f
    cand = jnp.where(g == m, key, 3.4e38)
    kmin = _reduce2(cand, jnp.min)
    mi = jnp.floor(kmin * (1.0 / 128.0))
    mj = kmin - mi * 128.0

    mi_s = mi[0, 0].astype(jnp.int32)
    h_e = jnp.clip(mi_s - 2, 0, H - 6) & ~1
    row0 = pl.multiple_of(h_e * 4, 8)
    win = pred_ref[0, 0, pl.ds(row0, 24), :]
    w3 = win.reshape(6, 4, W)
    xw0 = w3[:, 0, :]
    yw0 = w3[:, 1, :]
    xw1 = w3[:, 2, :]
    yw1 = w3[:, 3, :]

    gw = _giou(xw0, yw0, xw1, yw1, gx0, gy0, gx1, gy1)
    l1w = (jnp.abs(xw0 - gx0) + jnp.abs(yw0 - gy0)
           + jnp.abs(xw1 - gx1) + jnp.abs(yw1 - gy1)) * 0.25

    hw = (lax.broadcasted_iota(jnp.int32, (6, W), 0)
          + h_e).astype(jnp.float32)
    cw = lax.broadcasted_iota(jnp.int32, (6, W), 1).astype(jnp.float32)
    rmask = (hw >= mi - float(R_LO)) & (hw <= jnp.minimum(mi + float(R_HI), float(H - 1)))
    cmask = (cw >= mj - float(R_LO)) & (cw <= jnp.minimum(mj + float(R_HI), float(W - 1)))
    maskw = jnp.where(rmask & cmask, 1.0, 0.0)

    s_l1 = _reduce2(l1w * maskw, jnp.sum)
    s_g = _reduce2((1.0 - gw) * maskw, jnp.sum)
    s_c = _reduce2(maskw, jnp.sum)

    valid = jnp.where(m > POS_IOU_THR, 1.0, 0.0)

    lane = lax.broadcasted_iota(jnp.int32, (1, 128), 1)
    return (jnp.where(lane == 0, 1.0, 0.0) * s_l1
            + jnp.where(lane == 1, 1.0, 0.0) * s_g
            + jnp.where(lane == 2, 1.0, 0.0) * s_c) * valid


def _loss_kernel(labels_ref, p0, p1, p2, p3, p4, p5, p6, p7, gt_ref, out_ref):
    b = pl.program_id(0)
    m = pl.program_id(1)

    rows = []
    for i, p in enumerate((p0, p1, p2, p3, p4, p5, p6, p7)):
        base = (b * N + m * G + i) * 4
        rows.append(_one_gt(p, gt_ref[base + 0], gt_ref[base + 1],
                            gt_ref[base + 2], gt_ref[base + 3]))
    out_ref[...] = jnp.concatenate(rows, axis=0).reshape(1, G, 128)


def kernel(pred_boxes, gt_boxes, gt_labels):
    pred_t = jnp.moveaxis(pred_boxes, -1, 3).reshape(B, C, 4 * H, W)
    gt_flat = gt_boxes.reshape(-1)
    labels = gt_labels.astype(jnp.int32)

    pred_spec = [
        pl.BlockSpec((1, 1, 4 * H, W),
                     (lambda b, m, labels, i=i: (b, labels[b, m * G + i], 0, 0)))
        for i in range(G)
    ]

    out = pl.pallas_call(
        _loss_kernel,
        grid_spec=pltpu.PrefetchScalarGridSpec(
            num_scalar_prefetch=1,
            grid=(B, N // G),
            in_specs=pred_spec + [pl.BlockSpec(memory_space=pltpu.SMEM)],
            out_specs=pl.BlockSpec((1, G, 128),
                                   lambda b, m, labels: (b * (N // G) + m, 0, 0)),
        ),
        out_shape=jax.ShapeDtypeStruct((B * N // G, G, 128), jnp.float32),
        compiler_params=pltpu.CompilerParams(
            dimension_semantics=("parallel", "arbitrary"),
        ),
        name="gtclass_loc_loss",
    )(labels, *([pred_t] * G), gt_flat)

    l1_sum = jnp.sum(out[:, :, 0])
    g_sum = jnp.sum(out[:, :, 1])
    n_pos = jnp.sum(out[:, :, 2])
    denom = jnp.maximum(n_pos, 1.0)
    return LAMBDA_L1 * (l1_sum / denom) + LAMBDA_GIOU * (g_sum / denom)


# speedup vs baseline: 1.0667x; 1.0667x over previous
import jax
import jax.numpy as jnp
from jax import lax
from jax.experimental import pallas as pl
from jax.experimental.pallas import tpu as pltpu

B, C, H, W, N = 16, 80, 100, 100, 32
LAMBDA_L1, LAMBDA_GIOU, POS_IOU_THR = 1.0, 2.0, 0.3
R_LO, R_HI = 2, 1
G = 16


def _reduce2(x, op):
    r = op(x, axis=0, keepdims=True)
    return op(r, axis=1, keepdims=True)


def _one_gt(pred_ref, gx0, gy0, gx1, gy1):
    x0 = pred_ref[0, 0, 0::4, :]
    y0 = pred_ref[0, 0, 1::4, :]
    x1 = pred_ref[0, 0, 2::4, :]
    y1 = pred_ref[0, 0, 3::4, :]

    area_a = (x1 - x0) * (y1 - y0)
    area_b = (gx1 - gx0) * (gy1 - gy0)
    ltx = jnp.maximum(x0, gx0)
    lty = jnp.maximum(y0, gy0)
    rbx = jnp.minimum(x1, gx1)
    rby = jnp.minimum(y1, gy1)
    iw = jnp.maximum(rbx - ltx, 0.0)
    ih = jnp.maximum(rby - lty, 0.0)
    inter = iw * ih
    union = area_a + area_b - inter
    iou = inter / union
    cx = jnp.minimum(x0, gx0)
    cy = jnp.minimum(y0, gy0)
    dx = jnp.maximum(x1, gx1)
    dy = jnp.maximum(y1, gy1)
    area_c = (dx - cx) * (dy - cy)
    g = iou - (area_c - union) / area_c

    m = _reduce2(g, jnp.max)
    rows_f = lax.broadcasted_iota(jnp.int32, (H, W), 0).astype(jnp.float32)
    cols_f = lax.broadcasted_iota(jnp.int32, (H, W), 1).astype(jnp.float32)
    key = rows_f * 128.0 + cols_f
    cand = jnp.where(g == m, key, 3.4e38)
    kmin = _reduce2(cand, jnp.min)
    mi = jnp.floor(kmin * (1.0 / 128.0))
    mj = kmin - mi * 128.0

    rmask = (rows_f >= mi - float(R_LO)) & (rows_f <= jnp.minimum(mi + float(R_HI), float(H - 1)))
    cmask = (cols_f >= mj - float(R_LO)) & (cols_f <= jnp.minimum(mj + float(R_HI), float(W - 1)))
    mask = jnp.where(rmask & cmask, 1.0, 0.0)

    l1 = (jnp.abs(x0 - gx0) + jnp.abs(y0 - gy0)
          + jnp.abs(x1 - gx1) + jnp.abs(y1 - gy1)) * 0.25

    s_l1 = _reduce2(l1 * mask, jnp.sum)
    rn = jnp.minimum(mi + float(R_HI), float(H - 1)) - jnp.maximum(mi - float(R_LO), 0.0) + 1.0
    cn = jnp.minimum(mj + float(R_HI), float(W - 1)) - jnp.maximum(mj - float(R_LO), 0.0) + 1.0
    s_c = rn * cn
    s_g = s_c - _reduce2(g * mask, jnp.sum)

    valid = jnp.where(m > POS_IOU_THR, 1.0, 0.0)

    lane = lax.broadcasted_iota(jnp.int32, (1, 128), 1)
    return (jnp.where(lane == 0, 1.0, 0.0) * s_l1
            + jnp.where(lane == 1, 1.0, 0.0) * s_g
            + jnp.where(lane == 2, 1.0, 0.0) * s_c) * valid


def _loss_kernel(labels_ref, *refs):
    preds, gt_ref, out_ref = refs[:G], refs[G], refs[G + 1]
    s = pl.program_id(0)

    rows = []
    for i, p in enumerate(preds):
        base = (s * G + i) * 4
        rows.append(_one_gt(p, gt_ref[base + 0], gt_ref[base + 1],
                            gt_ref[base + 2], gt_ref[base + 3]))
    out_ref[...] = jnp.concatenate(rows, axis=0).reshape(1, G, 128)


def kernel(pred_boxes, gt_boxes, gt_labels):
    pred_t = jnp.moveaxis(pred_boxes, -1, 3).reshape(B, C, 4 * H, W)
    gt_flat = gt_boxes.reshape(-1)
    labels = gt_labels.astype(jnp.int32)

    labels_flat = labels.reshape(-1)

    pred_spec = [
        pl.BlockSpec((1, 1, 4 * H, W),
                     (lambda s, labels, i=i:
                      ((s * G + i) // N, labels[s * G + i], 0, 0)))
        for i in range(G)
    ]

    out = pl.pallas_call(
        _loss_kernel,
        grid_spec=pltpu.PrefetchScalarGridSpec(
            num_scalar_prefetch=1,
            grid=(B * N // G,),
            in_specs=pred_spec + [pl.BlockSpec(memory_space=pltpu.SMEM)],
            out_specs=pl.BlockSpec((1, G, 128),
                                   lambda s, labels: (s, 0, 0)),
        ),
        out_shape=jax.ShapeDtypeStruct((B * N // G, G, 128), jnp.float32),
        compiler_params=pltpu.CompilerParams(
            dimension_semantics=("arbitrary",),
        ),
        name="gtclass_loc_loss",
    )(labels_flat, *([pred_t] * G), gt_flat)

    l1_sum = jnp.sum(out[:, :, 0])
    g_sum = jnp.sum(out[:, :, 1])
    n_pos = jnp.sum(out[:, :, 2])
    denom = jnp.maximum(n_pos, 1.0)
    return LAMBDA_L1 * (l1_sum / denom) + LAMBDA_GIOU * (g_sum / denom)


# speedup vs baseline: 1.0930x; 1.0246x over previous
import jax
import jax.numpy as jnp
from jax import lax
from jax.experimental import pallas as pl
from jax.experimental.pallas import tpu as pltpu

B, C, H, W, N = 16, 80, 100, 100, 32
LAMBDA_L1, LAMBDA_GIOU, POS_IOU_THR = 1.0, 2.0, 0.3
R_LO, R_HI = 2, 1
G = 32


def _reduce2(x, op):
    r = op(x, axis=0, keepdims=True)
    return op(r, axis=1, keepdims=True)


def _one_gt(pred_ref, gx0, gy0, gx1, gy1):
    x0 = pred_ref[0, 0, 0::4, :]
    y0 = pred_ref[0, 0, 1::4, :]
    x1 = pred_ref[0, 0, 2::4, :]
    y1 = pred_ref[0, 0, 3::4, :]

    area_a = (x1 - x0) * (y1 - y0)
    area_b = (gx1 - gx0) * (gy1 - gy0)
    ltx = jnp.maximum(x0, gx0)
    lty = jnp.maximum(y0, gy0)
    rbx = jnp.minimum(x1, gx1)
    rby = jnp.minimum(y1, gy1)
    iw = jnp.maximum(rbx - ltx, 0.0)
    ih = jnp.maximum(rby - lty, 0.0)
    inter = iw * ih
    union = area_a + area_b - inter
    iou = inter / union
    cx = jnp.minimum(x0, gx0)
    cy = jnp.minimum(y0, gy0)
    dx = jnp.maximum(x1, gx1)
    dy = jnp.maximum(y1, gy1)
    area_c = (dx - cx) * (dy - cy)
    g = iou - (area_c - union) / area_c

    m = _reduce2(g, jnp.max)
    rows_f = lax.broadcasted_iota(jnp.int32, (H, W), 0).astype(jnp.float32)
    cols_f = lax.broadcasted_iota(jnp.int32, (H, W), 1).astype(jnp.float32)
    key = rows_f * 128.0 + cols_f
    cand = jnp.where(g == m, key, 3.4e38)
    kmin = _reduce2(cand, jnp.min)
    mi = jnp.floor(kmin * (1.0 / 128.0))
    mj = kmin - mi * 128.0

    rmask = (rows_f >= mi - float(R_LO)) & (rows_f <= jnp.minimum(mi + float(R_HI), float(H - 1)))
    cmask = (cols_f >= mj - float(R_LO)) & (cols_f <= jnp.minimum(mj + float(R_HI), float(W - 1)))
    mask = jnp.where(rmask & cmask, 1.0, 0.0)

    l1 = (jnp.abs(x0 - gx0) + jnp.abs(y0 - gy0)
          + jnp.abs(x1 - gx1) + jnp.abs(y1 - gy1)) * 0.25

    s_l1 = _reduce2(l1 * mask, jnp.sum)
    rn = jnp.minimum(mi + float(R_HI), float(H - 1)) - jnp.maximum(mi - float(R_LO), 0.0) + 1.0
    cn = jnp.minimum(mj + float(R_HI), float(W - 1)) - jnp.maximum(mj - float(R_LO), 0.0) + 1.0
    s_c = rn * cn
    s_g = s_c - _reduce2(g * mask, jnp.sum)

    valid = jnp.where(m > POS_IOU_THR, 1.0, 0.0)

    lane = lax.broadcasted_iota(jnp.int32, (1, 128), 1)
    return (jnp.where(lane == 0, 1.0, 0.0) * s_l1
            + jnp.where(lane == 1, 1.0, 0.0) * s_g
            + jnp.where(lane == 2, 1.0, 0.0) * s_c) * valid


def _loss_kernel(labels_ref, *refs):
    preds, gt_ref, out_ref = refs[:G], refs[G], refs[G + 1]
    s = pl.program_id(0)

    rows = []
    for i, p in enumerate(preds):
        base = (s * G + i) * 4
        rows.append(_one_gt(p, gt_ref[base + 0], gt_ref[base + 1],
                            gt_ref[base + 2], gt_ref[base + 3]))
    out_ref[...] = jnp.concatenate(rows, axis=0).reshape(1, G, 128)


def kernel(pred_boxes, gt_boxes, gt_labels):
    pred_t = jnp.moveaxis(pred_boxes, -1, 3).reshape(B, C, 4 * H, W)
    gt_flat = gt_boxes.reshape(-1)
    labels = gt_labels.astype(jnp.int32)

    labels_flat = labels.reshape(-1)

    pred_spec = [
        pl.BlockSpec((1, 1, 4 * H, W),
                     (lambda s, labels, i=i:
                      ((s * G + i) // N, labels[s * G + i], 0, 0)))
        for i in range(G)
    ]

    out = pl.pallas_call(
        _loss_kernel,
        grid_spec=pltpu.PrefetchScalarGridSpec(
            num_scalar_prefetch=1,
            grid=(B * N // G,),
            in_specs=pred_spec + [pl.BlockSpec(memory_space=pltpu.SMEM)],
            out_specs=pl.BlockSpec((1, G, 128),
                                   lambda s, labels: (s, 0, 0)),
        ),
        out_shape=jax.ShapeDtypeStruct((B * N // G, G, 128), jnp.float32),
        compiler_params=pltpu.CompilerParams(
            dimension_semantics=("arbitrary",),
        ),
        name="gtclass_loc_loss",
    )(labels_flat, *([pred_t] * G), gt_flat)

    l1_sum = jnp.sum(out[:, :, 0])
    g_sum = jnp.sum(out[:, :, 1])
    n_pos = jnp.sum(out[:, :, 2])
    denom = jnp.maximum(n_pos, 1.0)
    return LAMBDA_L1 * (l1_sum / denom) + LAMBDA_GIOU * (g_sum / denom)
